# baseline (speedup 1.0000x reference)
import jax
import jax.numpy as jnp
from jax import lax
from jax.experimental import pallas as pl
from jax.experimental.pallas import tpu as pltpu
from jax.experimental.pallas import tpu_sc as plsc

B = 16384
K = 16
NC = 2
NS = 16
NW = NC * NS
BPW = B // NW
GE = 8
NG = BPW // GE
DRAIN = GE * 2 * 2 * 8 * 16


def _sc_kernel(wt_hbm, ht_hbm, uidx_hbm, iidx_hbm, out_hbm,
               uidx_v, iidx_v, ubuf, vbuf, out_v, drain_v, sem):
    wid = lax.axis_index("c") * NS + lax.axis_index("s")
    base = wid * BPW

    pltpu.sync_copy(uidx_hbm.at[pl.ds(base, BPW)], uidx_v)
    pltpu.sync_copy(iidx_hbm.at[pl.ds(base, BPW)], iidx_v)

    lane = lax.iota(jnp.int32, 16)

    def body(g, _):
        uvals = plsc.load_gather(uidx_v, [g * GE + (lane & 7)])
        ivals = plsc.load_gather(iidx_v, [g * GE + (lane & 7)])
        for e in range(GE):
            bu = pl.multiple_of((uvals[e] >> 7) << 7, 128)
            bi = pl.multiple_of((ivals[e] >> 7) << 7, 128)
            su = (uvals[e] >> 4) & 7
            si = (ivals[e] >> 4) & 7
            for s in range(8):
                @pl.when(su == s)
                def _u(e=e, s=s, bu=bu):
                    for j in range(2):
                        pltpu.async_copy(
                            wt_hbm.at[j, :, pl.ds(bu, 128)]
                                  .at[:, pl.ds(s * 16, 16)],
                            ubuf.at[pl.ds((2 * e + j) * 8, 8), pl.ds(0, 16)],
                            sem)

                @pl.when(si == s)
                def _v(e=e, s=s, bi=bi):
                    for j in range(2):
                        pltpu.async_copy(
                            ht_hbm.at[j, :, pl.ds(bi, 128)]
                                  .at[:, pl.ds(s * 16, 16)],
                            vbuf.at[pl.ds((2 * e + j) * 8, 8), pl.ds(0, 16)],
                            sem)
        pltpu.make_async_copy(
            out_hbm.at[pl.ds(0, DRAIN)], drain_v, sem).wait()

        ucol = uvals & 15
        vcol = ivals & 15
        acc = jnp.zeros((16,), jnp.float32)
        for k in range(K):
            rowvec = 16 * (lane & 7) + k
            u = plsc.load_gather(ubuf, [rowvec, ucol])
            v = plsc.load_gather(vbuf, [rowvec, vcol])
            acc = acc + u * v
        plsc.store_scatter(out_v, [g * GE + (lane & 7)], acc, mask=lane < 8)
        return _

    lax.fori_loop(0, NG, body, None)

    pltpu.sync_copy(out_v, out_hbm.at[pl.ds(base, BPW)])


@jax.jit
def _run(wt, ht, uidx, iidx):
    mesh = plsc.VectorSubcoreMesh(core_axis_name="c", subcore_axis_name="s")
    fn = pl.kernel(
        _sc_kernel,
        mesh=mesh,
        compiler_params=pltpu.CompilerParams(needs_layout_passes=False),
        out_type=jax.ShapeDtypeStruct((B,), jnp.float32),
        scratch_types=[
            pltpu.VMEM((BPW,), jnp.int32),
            pltpu.VMEM((BPW,), jnp.int32),
            pltpu.VMEM((2 * GE * 8, 128), jnp.float32),
            pltpu.VMEM((2 * GE * 8, 128), jnp.float32),
            pltpu.VMEM((BPW,), jnp.float32),
            pltpu.VMEM((DRAIN,), jnp.float32),
            pltpu.SemaphoreType.DMA,
        ],
    )
    return fn(wt, ht, uidx, iidx)


def kernel(x, W, H):
    wt = W.T.reshape(2, 8, W.shape[0])
    ht = H.T.reshape(2, 8, H.shape[0])
    return _run(wt, ht, x[:, 0], x[:, 1])


# speedup vs baseline: 1.1661x; 1.1661x over previous
import jax
import jax.numpy as jnp
from jax import lax
from jax.experimental import pallas as pl
from jax.experimental.pallas import tpu as pltpu
from jax.experimental.pallas import tpu_sc as plsc

B = 16384
K = 16
NC = 2
NS = 16
NW = NC * NS
BPW = B // NW
GE = 4
NG = BPW // GE
DRAIN = GE * 2 * 2 * 8 * 16


def _sc_kernel(wt_hbm, ht_hbm, uidx_hbm, iidx_hbm, out_hbm,
               uidx_v, iidx_v, ubuf, vbuf, out_v, drain_v, sem0, sem1):
    wid = lax.axis_index("c") * NS + lax.axis_index("s")
    base = wid * BPW

    pltpu.sync_copy(uidx_hbm.at[pl.ds(base, BPW)], uidx_v)
    pltpu.sync_copy(iidx_hbm.at[pl.ds(base, BPW)], iidx_v)

    lane = lax.iota(jnp.int32, 16)

    def issue(gg, par):
        sem = sem0 if par == 0 else sem1
        uvals = plsc.load_gather(uidx_v, [gg * GE + (lane & 3)])
        ivals = plsc.load_gather(iidx_v, [gg * GE + (lane & 3)])
        for e in range(GE):
            bu = pl.multiple_of((uvals[e] >> 7) << 7, 128)
            bi = pl.multiple_of((ivals[e] >> 7) << 7, 128)
            su = (uvals[e] >> 4) & 7
            si = (ivals[e] >> 4) & 7
            for s in range(8):
                @pl.when(su == s)
                def _u(e=e, s=s, bu=bu):
                    for j in range(2):
                        pltpu.async_copy(
                            wt_hbm.at[j, :, pl.ds(bu, 128)]
                                  .at[:, pl.ds(s * 16, 16)],
                            ubuf.at[pl.ds(par * 64 + (2 * e + j) * 8, 8),
                                    pl.ds(0, 16)],
                            sem)

                @pl.when(si == s)
                def _v(e=e, s=s, bi=bi):
                    for j in range(2):
                        pltpu.async_copy(
                            ht_hbm.at[j, :, pl.ds(bi, 128)]
                                  .at[:, pl.ds(s * 16, 16)],
                            vbuf.at[pl.ds(par * 64 + (2 * e + j) * 8, 8),
                                    pl.ds(0, 16)],
                            sem)

    def process(g, par):
        sem = sem0 if par == 0 else sem1
        pltpu.make_async_copy(
            out_hbm.at[pl.ds(0, DRAIN)], drain_v, sem).wait()
        uvals = plsc.load_gather(uidx_v, [g * GE + (lane & 3)])
        ivals = plsc.load_gather(iidx_v, [g * GE + (lane & 3)])
        ucol = uvals & 15
        vcol = ivals & 15
        acc = jnp.zeros((16,), jnp.float32)
        for k in range(K):
            rowvec = par * 64 + 16 * (lane & 3) + k
            u = plsc.load_gather(ubuf, [rowvec, ucol])
            v = plsc.load_gather(vbuf, [rowvec, vcol])
            acc = acc + u * v
        plsc.store_scatter(out_v, [g * GE + (lane & 3)], acc, mask=lane < 4)

    issue(0, 0)

    def body(i, _):
        g = i * 2
        issue(g + 1, 1)
        process(g, 0)

        @pl.when(g + 2 < NG)
        def _():
            issue(g + 2, 0)
        process(g + 1, 1)
        return _

    lax.fori_loop(0, NG // 2, body, None)

    pltpu.sync_copy(out_v, out_hbm.at[pl.ds(base, BPW)])


@jax.jit
def _run(wt, ht, uidx, iidx):
    mesh = plsc.VectorSubcoreMesh(core_axis_name="c", subcore_axis_name="s")
    fn = pl.kernel(
        _sc_kernel,
        mesh=mesh,
        compiler_params=pltpu.CompilerParams(needs_layout_passes=False),
        out_type=jax.ShapeDtypeStruct((B,), jnp.float32),
        scratch_types=[
            pltpu.VMEM((BPW,), jnp.int32),
            pltpu.VMEM((BPW,), jnp.int32),
            pltpu.VMEM((2 * 2 * GE * 8, 128), jnp.float32),
            pltpu.VMEM((2 * 2 * GE * 8, 128), jnp.float32),
            pltpu.VMEM((BPW,), jnp.float32),
            pltpu.VMEM((DRAIN,), jnp.float32),
            pltpu.SemaphoreType.DMA,
            pltpu.SemaphoreType.DMA,
        ],
    )
    return fn(wt, ht, uidx, iidx)


def kernel(x, W, H):
    wt = W.T.reshape(2, 8, W.shape[0])
    ht = H.T.reshape(2, 8, H.shape[0])
    return _run(wt, ht, x[:, 0], x[:, 1])


# speedup vs baseline: 3.7690x; 3.2321x over previous
import jax
import jax.numpy as jnp
from jax import lax
from jax.experimental import pallas as pl
from jax.experimental.pallas import tpu as pltpu
from jax.experimental.pallas import tpu_sc as plsc

B = 16384
K = 16
NC = 2
NS = 16
NW = NC * NS
BPW = B // NW
NG = BPW // 16


def _sc_kernel(wt_hbm, ht_hbm, uidx_hbm, iidx_hbm, out_hbm,
               uidx_v, iidx_v, ubuf, vbuf, out_v, drain_v, sem):
    wid = lax.axis_index("c") * NS + lax.axis_index("s")
    base = wid * BPW

    pltpu.sync_copy(uidx_hbm.at[pl.ds(base, BPW)], uidx_v)
    pltpu.sync_copy(iidx_hbm.at[pl.ds(base, BPW)], iidx_v)

    lane = lax.iota(jnp.int32, 16)

    def group_body(g, _):
        sl = pl.ds(g * 16, 16)
        uvec = uidx_v[sl]
        ivec = iidx_v[sl]
        for e in range(16):
            bu = pl.multiple_of((uvec[e] >> 7) << 7, 128)
            bi = pl.multiple_of((ivec[e] >> 7) << 7, 128)
            su = (uvec[e] >> 6) & 1
            si = (ivec[e] >> 6) & 1
            for s in range(2):
                @pl.when(su == s)
                def _u(e=e, s=s, bu=bu):
                    for j in range(2):
                        pltpu.async_copy(
                            wt_hbm.at[j, :, pl.ds(bu, 128)]
                                  .at[:, pl.ds(s * 64, 64)],
                            ubuf.at[pl.ds((2 * e + j) * 8, 8), pl.ds(0, 64)],
                            sem)

                @pl.when(si == s)
                def _v(e=e, s=s, bi=bi):
                    for j in range(2):
                        pltpu.async_copy(
                            ht_hbm.at[j, :, pl.ds(bi, 128)]
                                  .at[:, pl.ds(s * 64, 64)],
                            vbuf.at[pl.ds((2 * e + j) * 8, 8), pl.ds(0, 64)],
                            sem)
        pltpu.make_async_copy(
            wt_hbm.at[0, :, pl.ds(0, 4096)], drain_v, sem).wait()
        acc = jnp.zeros((16,), jnp.float32)
        ucol = uvec & 63
        vcol = ivec & 63
        for k in range(K):
            rowvec = 16 * lane + k
            u = plsc.load_gather(ubuf, [rowvec, ucol])
            v = plsc.load_gather(vbuf, [rowvec, vcol])
            acc = acc + u * v
        out_v[sl] = acc
        return _

    lax.fori_loop(0, NG, group_body, None)

    pltpu.sync_copy(out_v, out_hbm.at[pl.ds(base, BPW)])


@jax.jit
def _run(wt, ht, uidx, iidx):
    mesh = plsc.VectorSubcoreMesh(core_axis_name="c", subcore_axis_name="s")
    fn = pl.kernel(
        _sc_kernel,
        mesh=mesh,
        compiler_params=pltpu.CompilerParams(needs_layout_passes=False),
        out_type=jax.ShapeDtypeStruct((B,), jnp.float32),
        scratch_types=[
            pltpu.VMEM((BPW,), jnp.int32),
            pltpu.VMEM((BPW,), jnp.int32),
            pltpu.VMEM((256, 128), jnp.float32),
            pltpu.VMEM((256, 128), jnp.float32),
            pltpu.VMEM((BPW,), jnp.float32),
            pltpu.VMEM((8, 4096), jnp.float32),
            pltpu.SemaphoreType.DMA,
        ],
    )
    return fn(wt, ht, uidx, iidx)


def kernel(x, W, H):
    wt = W.T.reshape(2, 8, W.shape[0])
    ht = H.T.reshape(2, 8, H.shape[0])
    return _run(wt, ht, x[:, 0], x[:, 1])


# speedup vs baseline: 4.6370x; 1.2303x over previous
import jax
import jax.numpy as jnp
from jax import lax
from jax.experimental import pallas as pl
from jax.experimental.pallas import tpu as pltpu
from jax.experimental.pallas import tpu_sc as plsc

B = 16384
K = 16
NC = 2
NS = 16
NW = NC * NS
BPW = B // NW
NG = BPW // 16


def _sc_kernel(wt_hbm, ht_hbm, uidx_hbm, iidx_hbm, out_hbm,
               uidx_v, iidx_v, ubuf, vbuf, out_v, sem):
    wid = lax.axis_index("c") * NS + lax.axis_index("s")
    base = wid * BPW

    pltpu.sync_copy(uidx_hbm.at[pl.ds(base, BPW)], uidx_v)
    pltpu.sync_copy(iidx_hbm.at[pl.ds(base, BPW)], iidx_v)

    lane = lax.iota(jnp.int32, 16)

    def group_body(g, _):
        sl = pl.ds(g * 16, 16)
        uvec = uidx_v[sl]
        ivec = iidx_v[sl]
        copies = []
        for e in range(16):
            bu = pl.multiple_of((uvec[e] >> 7) << 7, 128)
            bi = pl.multiple_of((ivec[e] >> 7) << 7, 128)
            for j in range(2):
                copies.append(pltpu.async_copy(
                    wt_hbm.at[j, :, pl.ds(bu, 128)],
                    ubuf.at[pl.ds((2 * e + j) * 8, 8), :], sem))
                copies.append(pltpu.async_copy(
                    ht_hbm.at[j, :, pl.ds(bi, 128)],
                    vbuf.at[pl.ds((2 * e + j) * 8, 8), :], sem))
        for cp in copies:
            cp.wait()
        acc = jnp.zeros((16,), jnp.float32)
        ucol = uvec & 127
        vcol = ivec & 127
        for k in range(K):
            rowvec = 16 * lane + k
            u = plsc.load_gather(ubuf, [rowvec, ucol])
            v = plsc.load_gather(vbuf, [rowvec, vcol])
            acc = acc + u * v
        out_v[sl] = acc
        return _

    lax.fori_loop(0, NG, group_body, None)

    pltpu.sync_copy(out_v, out_hbm.at[pl.ds(base, BPW)])


@jax.jit
def _run(wt, ht, uidx, iidx):
    mesh = plsc.VectorSubcoreMesh(core_axis_name="c", subcore_axis_name="s")
    fn = pl.kernel(
        _sc_kernel,
        mesh=mesh,
        compiler_params=pltpu.CompilerParams(needs_layout_passes=False),
        out_type=jax.ShapeDtypeStruct((B,), jnp.float32),
        scratch_types=[
            pltpu.VMEM((BPW,), jnp.int32),
            pltpu.VMEM((BPW,), jnp.int32),
            pltpu.VMEM((256, 128), jnp.float32),
            pltpu.VMEM((256, 128), jnp.float32),
            pltpu.VMEM((BPW,), jnp.float32),
            pltpu.SemaphoreType.DMA,
        ],
    )
    return fn(wt, ht, uidx, iidx)


def kernel(x, W, H):
    wt = W.T.reshape(2, 8, W.shape[0])
    ht = H.T.reshape(2, 8, H.shape[0])
    return _run(wt, ht, x[:, 0], x[:, 1])
